# baseline (speedup 1.0000x reference)
import functools

import jax
import jax.numpy as jnp
from jax import lax
from jax.experimental import pallas as pl
from jax.experimental.pallas import tpu as pltpu
from jax.experimental.pallas import tpu_sc as plsc

E_DIM = 64
NC = 2
NS = 16
L = 16
NW = NC * NS
BLK = 128
NGRP = BLK // L


def _rsqrt(x):
    i = plsc.bitcast(x, jnp.int32)
    i = jnp.int32(0x5F3759DF) - lax.shift_right_logical(i, 1)
    y = plsc.bitcast(i, jnp.float32)
    half_x = x * jnp.float32(0.5)
    for _ in range(3):
        y = y * (jnp.float32(1.5) - half_x * y * y)
    return y


def _make_sc_call(batch):
    per_w = batch // NW
    nblk = per_w // BLK
    mesh = plsc.VectorSubcoreMesh(
        core_axis_name="c", subcore_axis_name="s", num_cores=NC, num_subcores=NS
    )

    @functools.partial(
        pl.kernel,
        out_type=jax.ShapeDtypeStruct((batch,), jnp.float32),
        mesh=mesh,
        compiler_params=pltpu.CompilerParams(
            needs_layout_passes=False, use_tc_tiling_on_sc=False
        ),
        scratch_types=[
            pltpu.VMEM((3, nblk, BLK), jnp.int32),
            pltpu.VMEM((4, 3, BLK, E_DIM), jnp.float32),
            pltpu.VMEM((per_w,), jnp.float32),
            pltpu.SemaphoreType.DMA,
            pltpu.SemaphoreType.DMA,
            pltpu.SemaphoreType.DMA,
            pltpu.SemaphoreType.DMA,
        ],
    )
    def sc_call(
        idx_hbm, ent_hbm, rel_hbm, out_hbm, idx_v, rows, out_v, sem0, sem1, sem2, sem3
    ):
        sems = (sem0, sem1, sem2, sem3)
        wid = lax.axis_index("s") * NC + lax.axis_index("c")
        base = wid * per_w

        for r in range(3):
            pltpu.sync_copy(idx_hbm.at[r, pl.ds(wid * nblk, nblk)], idx_v.at[r])

        def issue(j, slot):
            cps = []
            for r, tab in ((0, ent_hbm), (1, rel_hbm), (2, ent_hbm)):
                cps.append(
                    pltpu.async_copy(
                        tab.at[idx_v.at[r, j]],
                        rows.at[slot, r],
                        sems[slot],
                    )
                )
            return cps

        lane = lax.iota(jnp.int32, L)

        def compute(j, slot):
            sbuf = rows.at[slot, 0]
            pbuf = rows.at[slot, 1]
            obuf = rows.at[slot, 2]

            def group(g, _):
                rowid = lane + g * jnp.int32(L)
                z = jnp.zeros((L,), jnp.float32)
                dot, ns, np_, no = z, z, z, z
                for k in range(E_DIM):
                    col = jnp.full((L,), k, jnp.int32)
                    sv = plsc.load_gather(sbuf, [rowid, col])
                    pv = plsc.load_gather(pbuf, [rowid, col])
                    ov = plsc.load_gather(obuf, [rowid, col])
                    sp = sv * pv
                    dot = dot + sp * ov
                    ns = ns + sv * sv
                    np_ = np_ + pv * pv
                    no = no + ov * ov
                eps = jnp.float32(1e-12)
                prod = (
                    jnp.maximum(ns, eps)
                    * jnp.maximum(np_, eps)
                    * jnp.maximum(no, eps)
                )
                out_v[pl.ds(j * BLK + g * L, L)] = dot * _rsqrt(prod)
                return 0

            lax.fori_loop(0, NGRP, group, 0)

        inflight = [issue(j, j) for j in range(nblk)]
        for j in range(nblk):
            for c in inflight[j]:
                c.wait()
            compute(j, j)

        pltpu.sync_copy(out_v, out_hbm.at[pl.ds(base, per_w)])

    return sc_call


@jax.jit
def kernel(inputs, entity_table, rel_table):
    batch = inputs.shape[0]
    per_w = batch // NW
    nblk = per_w // BLK
    idx = jnp.transpose(inputs).reshape(3, NW * nblk, BLK)
    scores = _make_sc_call(batch)(idx, entity_table, rel_table)
    return scores.reshape(batch, 1)


# speedup vs baseline: 1.0295x; 1.0295x over previous
import functools

import jax
import jax.numpy as jnp
from jax import lax
from jax.experimental import pallas as pl
from jax.experimental.pallas import tpu as pltpu
from jax.experimental.pallas import tpu_sc as plsc

E_DIM = 64
PAIR = 2 * E_DIM
NC = 2
NS = 16
L = 16
NW = NC * NS
BLK = 128
NGRP = BLK // L


def _rsqrt(x):
    i = plsc.bitcast(x, jnp.int32)
    i = jnp.int32(0x5F3759DF) - lax.shift_right_logical(i, 1)
    y = plsc.bitcast(i, jnp.float32)
    half_x = x * jnp.float32(0.5)
    for _ in range(3):
        y = y * (jnp.float32(1.5) - half_x * y * y)
    return y


def _make_sc_call(batch):
    per_w = batch // NW
    nblk = per_w // BLK
    mesh = plsc.VectorSubcoreMesh(
        core_axis_name="c", subcore_axis_name="s", num_cores=NC, num_subcores=NS
    )

    @functools.partial(
        pl.kernel,
        out_type=jax.ShapeDtypeStruct((batch,), jnp.float32),
        mesh=mesh,
        compiler_params=pltpu.CompilerParams(
            needs_layout_passes=False, use_tc_tiling_on_sc=False
        ),
        scratch_types=[
            pltpu.VMEM((3, nblk, BLK), jnp.int32),
            pltpu.VMEM((3, nblk, BLK), jnp.int32),
            pltpu.VMEM((2, 3, BLK, PAIR), jnp.float32),
            pltpu.VMEM((per_w,), jnp.float32),
            pltpu.SemaphoreType.DMA,
            pltpu.SemaphoreType.DMA,
        ],
    )
    def sc_call(
        idx_hbm, ent_hbm, rel_hbm, out_hbm, idx_v, idx2_v, rows, out_v, sem0, sem1
    ):
        sems = (sem0, sem1)
        wid = lax.axis_index("s") * NC + lax.axis_index("c")
        base = wid * per_w

        for r in range(3):
            pltpu.sync_copy(idx_hbm.at[r, pl.ds(wid * nblk, nblk)], idx_v.at[r])

        for r in range(3):
            for j in range(nblk):
                for i in range(BLK // L):
                    v = idx_v.at[r, j][pl.ds(i * L, L)]
                    idx2_v.at[r, j][pl.ds(i * L, L)] = lax.shift_right_logical(v, 1)

        def issue(j, slot):
            cps = []
            for r, tab in ((0, ent_hbm), (1, rel_hbm), (2, ent_hbm)):
                cps.append(
                    pltpu.async_copy(
                        tab.at[idx2_v.at[r, j]],
                        rows.at[slot, r],
                        sems[slot],
                    )
                )
            return cps

        lane = lax.iota(jnp.int32, L)

        def compute(j, slot):
            sbuf = rows.at[slot, 0]
            pbuf = rows.at[slot, 1]
            obuf = rows.at[slot, 2]

            def group(g, _):
                rowid = lane + g * jnp.int32(L)
                cols = []
                for r in range(3):
                    iv = idx_v.at[r, j][pl.ds(g * L, L)]
                    cols.append(
                        lax.shift_left(iv & jnp.int32(1), jnp.int32(6))
                    )
                scol, pcol, ocol = cols
                z = jnp.zeros((L,), jnp.float32)
                dot, ns, np_, no = z, z, z, z
                for k in range(E_DIM):
                    kv = jnp.int32(k)
                    sv = plsc.load_gather(sbuf, [rowid, scol + kv])
                    pv = plsc.load_gather(pbuf, [rowid, pcol + kv])
                    ov = plsc.load_gather(obuf, [rowid, ocol + kv])
                    sp = sv * pv
                    dot = dot + sp * ov
                    ns = ns + sv * sv
                    np_ = np_ + pv * pv
                    no = no + ov * ov
                eps = jnp.float32(1e-12)
                prod = (
                    jnp.maximum(ns, eps)
                    * jnp.maximum(np_, eps)
                    * jnp.maximum(no, eps)
                )
                out_v[pl.ds(j * BLK + g * L, L)] = dot * _rsqrt(prod)
                return 0

            lax.fori_loop(0, NGRP, group, 0)

        inflight = issue(0, 0)
        for j in range(nblk):
            slot = j % 2
            for c in inflight:
                c.wait()
            if j + 1 < nblk:
                inflight = issue(j + 1, (j + 1) % 2)
            compute(j, slot)

        pltpu.sync_copy(out_v, out_hbm.at[pl.ds(base, per_w)])

    return sc_call


@jax.jit
def kernel(inputs, entity_table, rel_table):
    batch = inputs.shape[0]
    per_w = batch // NW
    nblk = per_w // BLK
    idx = jnp.transpose(inputs).reshape(3, NW * nblk, BLK)
    ent2 = jnp.reshape(entity_table, (entity_table.shape[0] // 2, PAIR))
    rel2 = jnp.reshape(rel_table, (rel_table.shape[0] // 2, PAIR))
    scores = _make_sc_call(batch)(idx, ent2, rel2)
    return scores.reshape(batch, 1)
